# baseline (speedup 1.0000x reference)
import functools

import jax
import jax.numpy as jnp
from jax import lax
from jax.experimental import pallas as pl
from jax.experimental.pallas import tpu as pltpu
from jax.experimental.pallas import tpu_sc as plsc

NC = 2
NS = 16
LN = 16
NW = NC * NS
H = 128
F32 = jnp.float32

_MESH = plsc.VectorSubcoreMesh(core_axis_name="c", subcore_axis_name="s")
_SC_PARAMS = pltpu.CompilerParams(use_tc_tiling_on_sc=False)


def _zero_vmem(buf, nrows, width):
    z = jnp.zeros((LN,), F32)

    def body(r, _):
        for t in range(width // LN):
            buf[r, pl.ds(LN * t, LN)] = z
        return 0

    lax.fori_loop(0, nrows, body, 0)


def _fill_shared(zb, shared, start, nrows):
    start = pl.multiple_of(start, 8)
    off = 0
    while off < nrows:
        sz = min(128, nrows - off)
        pltpu.sync_copy(zb.at[pl.ds(0, sz)], shared.at[pl.ds(start + off, sz)])
        off += sz


@functools.cache
def _encoder_kernel(npad, nch, nrg):
    rg_per = nrg // NW

    @functools.partial(
        pl.kernel,
        out_type=(jax.ShapeDtypeStruct((npad, H), F32),
                  jax.ShapeDtypeStruct((nrg, H), F32)),
        mesh=_MESH,
        compiler_params=_SC_PARAMS,
        scratch_types=[
            pltpu.VMEM((9, nch, 128), jnp.int32),
            pltpu.VMEM((rg_per,), jnp.int32),
            pltpu.VMEM((nch * 128, H), F32),
            pltpu.VMEM((128, H), F32),
            pltpu.VMEM((128, H), F32),
            pltpu.VMEM((rg_per, H), F32),
            pltpu.SemaphoreType.DMA,
            pltpu.SemaphoreType.DMA,
        ],
    )
    def k(ae_h, nf_h, re_h, rf_h, x0_h, rg0_h, nfv, rfv, buf, e0, e1, rbuf,
          sem_a, sem_b):
        c = lax.axis_index("c")
        s = lax.axis_index("s")
        w = s * NC + c
        pltpu.sync_copy(nf_h.at[w], nfv)
        for ki in range(1, 9):
            off = jnp.full((LN,), 100 * ki, jnp.int32)

            def ob(r, _, ki=ki, off=off):
                for t in range(128 // LN):
                    sl = pl.ds(LN * t, LN)
                    nfv[ki, r, sl] = nfv[ki, r, sl] + off
                return 0

            lax.fori_loop(0, nch, ob, 0)
        units = [(ki, j) for ki in range(9) for j in range(nch)]
        es = (e0, e1)
        sems = (sem_a, sem_b)

        def fire(u, p):
            ki, j = units[u]
            pltpu.async_copy(ae_h.at[nfv.at[ki, j]], es[p], sems[p])

        fire(0, 0)
        for u, (ki, j) in enumerate(units):
            p = u % 2
            if u + 1 < len(units):
                fire(u + 1, 1 - p)
            pltpu.make_async_copy(ae_h.at[nfv.at[ki, j]], es[p],
                                  sems[p]).wait()

            def accum(r, _, ep=es[p], j=j, first=(ki == 0)):
                for t in range(H // LN):
                    sl = pl.ds(LN * t, LN)
                    if first:
                        buf[j * 128 + r, sl] = ep[r, sl]
                    else:
                        buf[j * 128 + r, sl] = buf[j * 128 + r, sl] + ep[r, sl]
                return 0

            lax.fori_loop(0, 128, accum, 0, unroll=2)
        pltpu.sync_copy(
            buf, x0_h.at[pl.ds(pl.multiple_of(w * (nch * 128), 8),
                               nch * 128)])
        pltpu.sync_copy(rf_h.at[w], rfv)
        pltpu.async_copy(re_h.at[rfv], rbuf, sem_a).wait()
        pltpu.sync_copy(rbuf,
                        rg0_h.at[pl.ds(pl.multiple_of(w * rg_per, 8), rg_per)])

    return k


HH = H // 2


@functools.cache
def _edge_agg_kernel(n_nodes, nch, hsplit=4):
    hw = H // hsplit
    npo = -(-n_nodes // 128) * 128
    acc_rows = npo if npo > n_nodes else npo + 128
    rows_z = acc_rows // NS
    rows_o = npo // NS

    npairs = nch // 2

    def k(*refs):
        parts = tuple((refs[p], refs[hsplit + p], refs[2 * hsplit + p])
                      for p in range(hsplit))
        src_h, dst_h, f0_h, f1_h, f2_h = refs[3 * hsplit:3 * hsplit + 5]
        (out_h, srcv, dstv, i01v, f2v, a0, a1, a2, b0, b1, b2, zb, acc,
         sem_a, sem_b) = refs[3 * hsplit + 5:]
        c = lax.axis_index("c")
        s = lax.axis_index("s")
        w = s * NC + c
        slots = ((a0, a1, a2, sem_a), (b0, b1, b2, sem_b))
        pltpu.sync_copy(src_h.at[w], srcv)
        pltpu.sync_copy(dst_h.at[w], dstv)
        pltpu.sync_copy(f0_h.at[w], i01v)
        pltpu.sync_copy(f1_h.at[w], f2v)
        hundred = jnp.full((LN,), 100, jnp.int32)

        def comb(r, _):
            for t in range(128 // LN):
                sl = pl.ds(LN * t, LN)
                i01v[r, sl] = i01v[r, sl] * hundred + f2v[r, sl]
            return 0

        lax.fori_loop(0, nch, comb, 0)
        pltpu.sync_copy(f2_h.at[w], f2v)
        _zero_vmem(zb, 128, hw)
        zv = jnp.zeros((LN,), F32)
        for part, (t01_h, t2_h, x_h) in enumerate(parts):
            _fill_shared(zb, acc, s * rows_z, rows_z)
            plsc.subcore_barrier()

            def fire(j, slot, x_h=x_h, t01_h=t01_h, t2_h=t2_h):
                g0, g1, g2, sem = slot
                pltpu.async_copy(t01_h.at[i01v.at[j]], g0, sem)
                pltpu.async_copy(t2_h.at[f2v.at[j]], g1, sem)
                pltpu.async_copy(x_h.at[srcv.at[j]], g2, sem)

            def proc(j, slot, x_h=x_h, t01_h=t01_h, t2_h=t2_h):
                g0, g1, g2, sem = slot
                pltpu.make_async_copy(t01_h.at[i01v.at[j]], g0, sem).wait()
                pltpu.make_async_copy(t2_h.at[f2v.at[j]], g1, sem).wait()
                pltpu.make_async_copy(x_h.at[srcv.at[j]], g2, sem).wait()

                def rel(r, _):
                    for t in range(hw // LN):
                        sl = pl.ds(LN * t, LN)
                        g0[r, sl] = jnp.maximum(
                            g0[r, sl] + g1[r, sl] + g2[r, sl], zv)
                    return 0

                lax.fori_loop(0, 128, rel, 0, unroll=4)
                pltpu.sync_copy(g0, acc.at[dstv.at[j]], add=True)

            fire(0, slots[0])

            def pair(i, _):
                j0 = 2 * i
                fire(j0 + 1, slots[1])
                proc(j0, slots[0])

                @pl.when(i + 1 < npairs)
                def _():
                    fire(j0 + 2, slots[0])

                proc(j0 + 1, slots[1])
                return 0

            lax.fori_loop(0, npairs, pair, 0)
            plsc.subcore_barrier()
            ro = pl.multiple_of(s * rows_o, 8)
            pltpu.sync_copy(acc.at[pl.ds(ro, rows_o)],
                            out_h.at[c, part, pl.ds(ro, rows_o)])
            if part + 1 < hsplit:
                plsc.subcore_barrier()

    return functools.partial(
        pl.kernel,
        out_type=jax.ShapeDtypeStruct((NC, hsplit, npo, hw), F32),
        mesh=_MESH,
        compiler_params=_SC_PARAMS,
        scratch_types=[
            pltpu.VMEM((nch, 128), jnp.int32),
            pltpu.VMEM((nch, 128), jnp.int32),
            pltpu.VMEM((nch, 128), jnp.int32),
            pltpu.VMEM((nch, 128), jnp.int32),
            pltpu.VMEM((128, hw), F32),
            pltpu.VMEM((128, hw), F32),
            pltpu.VMEM((128, hw), F32),
            pltpu.VMEM((128, hw), F32),
            pltpu.VMEM((128, hw), F32),
            pltpu.VMEM((128, hw), F32),
            pltpu.VMEM((128, hw), F32),
            pltpu.VMEM_SHARED((acc_rows, hw), F32),
            pltpu.SemaphoreType.DMA,
            pltpu.SemaphoreType.DMA,
        ],
    )(k)


@functools.cache
def _seg_sum_kernel(nch, nseg, hsplit=2):
    hw = H // hsplit
    npo = -(-nseg // 128) * 128
    acc_rows = npo if npo > nseg else npo + 128
    rows_z = acc_rows // NS
    rows_o = npo // NS

    npairs = nch // 2

    def k(*refs):
        tbls = refs[:hsplit]
        g_h, s_h, sum_h, cnt_h = refs[hsplit:hsplit + 4]
        (gv, sv, bufa, bufb, zb, onesb, acc, cacc,
         sem_a, sem_b) = refs[hsplit + 4:]
        c = lax.axis_index("c")
        s = lax.axis_index("s")
        w = s * NC + c
        slots = ((bufa, sem_a), (bufb, sem_b))
        pltpu.sync_copy(g_h.at[w], gv)
        pltpu.sync_copy(s_h.at[w], sv)
        _zero_vmem(zb, 128, hw)
        _zero_vmem(onesb, 128, LN)
        _fill_shared(onesb, cacc, s * rows_z, rows_z)
        one = jnp.ones((LN,), F32)

        def setone(r, _):
            onesb[r, pl.ds(0, LN)] = one
            return 0

        lax.fori_loop(0, 128, setone, 0)
        for part, tbl_h in enumerate(tbls):
            _fill_shared(zb, acc, s * rows_z, rows_z)
            plsc.subcore_barrier()

            def fire(j, slot, tbl_h=tbl_h):
                pltpu.async_copy(tbl_h.at[gv.at[j]], slot[0], slot[1])

            def proc(j, slot, tbl_h=tbl_h, part=part):
                pltpu.make_async_copy(tbl_h.at[gv.at[j]], slot[0],
                                      slot[1]).wait()
                pltpu.sync_copy(slot[0], acc.at[sv.at[j]], add=True)
                if part == 0:
                    pltpu.sync_copy(onesb, cacc.at[sv.at[j]], add=True)

            fire(0, slots[0])

            def pair(i, _):
                j0 = 2 * i
                fire(j0 + 1, slots[1])
                proc(j0, slots[0])

                @pl.when(i + 1 < npairs)
                def _():
                    fire(j0 + 2, slots[0])

                proc(j0 + 1, slots[1])
                return 0

            lax.fori_loop(0, npairs, pair, 0)
            plsc.subcore_barrier()
            ro = pl.multiple_of(s * rows_o, 8)
            pltpu.sync_copy(acc.at[pl.ds(ro, rows_o)],
                            sum_h.at[c, part, pl.ds(ro, rows_o)])
            if part == 0:
                pltpu.sync_copy(cacc.at[pl.ds(ro, rows_o)],
                                cnt_h.at[c, pl.ds(ro, rows_o)])
            if part + 1 < hsplit:
                plsc.subcore_barrier()

    return functools.partial(
        pl.kernel,
        out_type=(jax.ShapeDtypeStruct((NC, hsplit, npo, hw), F32),
                  jax.ShapeDtypeStruct((NC, npo, LN), F32)),
        mesh=_MESH,
        compiler_params=_SC_PARAMS,
        scratch_types=[
            pltpu.VMEM((nch, 128), jnp.int32),
            pltpu.VMEM((nch, 128), jnp.int32),
            pltpu.VMEM((128, hw), F32),
            pltpu.VMEM((128, hw), F32),
            pltpu.VMEM((128, hw), F32),
            pltpu.VMEM((128, LN), F32),
            pltpu.VMEM_SHARED((acc_rows, hw), F32),
            pltpu.VMEM_SHARED((acc_rows, LN), F32),
            pltpu.SemaphoreType.DMA,
            pltpu.SemaphoreType.DMA,
        ],
    )(k)


def _pair_tc(t0, t1):
    def body(t0_r, t1_r, o_r):
        i = pl.program_id(0)
        for r in range(4):
            o_r[pl.ds(100 * r, 100)] = t0_r[pl.ds(4 * i + r, 1)] + t1_r[...]

    return pl.pallas_call(
        body,
        grid=(25,),
        in_specs=[pl.BlockSpec((100, H), lambda i: (0, 0)),
                  pl.BlockSpec((100, H), lambda i: (0, 0))],
        out_specs=pl.BlockSpec((400, H), lambda i: (i, 0)),
        out_shape=jax.ShapeDtypeStruct((100 * 100, H), F32),
    )(t0, t1)


def _gine_tc(x, aggp, eps, W1, b1, g1, be1, W2, b2, og, ob):
    n = x.shape[0]

    def body(x_r, a_r, e_r, w1_r, b1_r, g1_r, be1_r, w2_r, b2_r, og_r, ob_r,
             o_r):
        z = (1.0 + e_r[0, 0]) * x_r[...] + a_r[0] + a_r[1]
        h = jnp.dot(z, w1_r[...], preferred_element_type=F32) + b1_r[...]
        m = jnp.mean(h, 0, keepdims=True)
        v = jnp.mean((h - m) ** 2, 0, keepdims=True)
        h = jnp.maximum(g1_r[...] * (h - m) / jnp.sqrt(v + 1e-5) + be1_r[...],
                        0.0)
        h2 = jnp.dot(h, w2_r[...], preferred_element_type=F32) + b2_r[...]
        m2 = jnp.mean(h2, 0, keepdims=True)
        v2 = jnp.mean((h2 - m2) ** 2, 0, keepdims=True)
        o_r[...] = jnp.maximum(
            og_r[...] * (h2 - m2) / jnp.sqrt(v2 + 1e-5) + ob_r[...], 0.0)

    return pl.pallas_call(
        body, out_shape=jax.ShapeDtypeStruct((n, H), F32))(
            x, aggp, eps, W1, b1, g1, be1, W2, b2, og, ob)


def _segmean_proj_tc(base, sums, cnt, W, b, add_base):
    n = sums.shape[1]

    def body(base_r, s_r, c_r, w_r, b_r, o_r):
        cn = jnp.maximum((c_r[0] + c_r[1])[:, 0:1], 1.0)
        sm = (s_r[0] + s_r[1]) / cn
        proj = jnp.maximum(
            jnp.dot(sm, w_r[...], preferred_element_type=F32) + b_r[...], 0.0)
        o_r[...] = base_r[...] + proj

    def body_mul(base_r, s_r, c_r, w_r, b_r, o_r):
        pass

    del body_mul, add_base
    return pl.pallas_call(
        body, out_shape=jax.ShapeDtypeStruct((n, H), F32))(
            base, sums, cnt, W, b)


def _readout_tc(x, batch2, rg0, rg_hi2, rg_num2, al_W, al_b, rl_W, rl_b,
                lin_W, lin_b):
    n = x.shape[0]
    nrg = rg0.shape[0]
    g = 64
    out_d = lin_W.shape[1]

    def body(x_r, b_r, r0_r, hi_r, num_r, alw_r, alb_r, rlw_r, rlb_r, lw_r,
             lb_r, o_r):
        dn = (((0,), (0,)), ((), ()))
        bo = (b_r[...] == lax.broadcasted_iota(jnp.int32, (n, g), 1)
              ).astype(F32)
        xs = lax.dot_general(bo, x_r[...], dn, preferred_element_type=F32)
        cnt = lax.dot_general(bo, jnp.ones((n, 1), F32), dn,
                              preferred_element_type=F32)
        xm = xs / jnp.maximum(cnt, 1.0)
        xg = jnp.dot(xm, alw_r[...], preferred_element_type=F32) + alb_r[...]
        ri = lax.broadcasted_iota(jnp.int32, (nrg, g), 0)
        hi = hi_r[...]
        lo = hi - num_r[...]
        to = ((ri >= lo) & (ri < hi)).astype(F32)
        rs = lax.dot_general(to, r0_r[...], dn, preferred_element_type=F32)
        rcnt = lax.dot_general(to, jnp.ones((nrg, 1), F32), dn,
                               preferred_element_type=F32)
        rgm = rs / jnp.maximum(rcnt, 1.0)
        rgg = jnp.dot(rgm, rlw_r[...], preferred_element_type=F32) + rlb_r[...]
        act = jnp.maximum(xg + rgg, 0.0)
        o_r[...] = jnp.dot(act, lw_r[...], preferred_element_type=F32) + lb_r[...]

    return pl.pallas_call(
        body, out_shape=jax.ShapeDtypeStruct((g, out_d), F32))(
            x, batch2, rg0, rg_hi2, rg_num2, al_W, al_b, rl_W, rl_b, lin_W,
            lin_b)


def _merge_parts(a, m):
    return jnp.concatenate([a[:, i] for i in range(a.shape[1])],
                           axis=-1)[:, :m]


def _hparts(a, hs):
    hw = H // hs
    return tuple(a[:, i * hw:(i + 1) * hw] for i in range(hs))


def _prep_idx(a, fill):
    m = a.shape[0]
    nch = -(-m // (NW * 128))
    nch += nch % 2
    ap = jnp.pad(a.astype(jnp.int32), (0, NW * nch * 128 - m),
                 constant_values=fill)
    return ap.reshape(NW, nch, 128), nch


def kernel(node_feat, edge_index, edge_feat, batch, rg_edge_index_0,
           mapping_0, rg_num_atoms_0, rg_atom_features_0, params):
    n = node_feat.shape[0]
    nrg = rg_atom_features_0.shape[0]
    g = rg_num_atoms_0.shape[0]
    r2 = lambda v: v.reshape(1, -1)

    src_t, ech = _prep_idx(edge_index[0], 0)
    dst_t, _ = _prep_idx(edge_index[1], n)
    f0_t, _ = _prep_idx(edge_feat[:, 0], 0)
    f1_t, _ = _prep_idx(edge_feat[:, 1], 0)
    f2_t, _ = _prep_idx(edge_feat[:, 2], 0)
    row_g, mch = _prep_idx(mapping_0[0], 0)
    col_s, _ = _prep_idx(mapping_0[1], nrg)
    col_g, _ = _prep_idx(mapping_0[1], 0)
    row_s, _ = _prep_idx(mapping_0[0], n)
    s2_g, rch = _prep_idx(rg_edge_index_0[0], 0)
    d2_s, _ = _prep_idx(rg_edge_index_0[1], nrg)

    nch_n = -(-n // (NW * 128))
    npad = NW * nch_n * 128
    nf_t = jnp.pad(node_feat.T.astype(jnp.int32), ((0, 0), (0, npad - n))
                   ).reshape(9, NW, nch_n, 128).transpose(1, 0, 2, 3)
    ae = params["atom_emb"].reshape(900, H)
    rf_t = rg_atom_features_0.astype(jnp.int32).reshape(NW, nrg // NW)

    x0p, rg0 = _encoder_kernel(npad, nch_n, nrg)(ae, nf_t, params["rg_emb"],
                                                 rf_t)
    x = x0p[:n]

    for lp in params["layers"]:
        be = lp["bond_emb"]
        t01 = _pair_tc(be[0], be[1])
        aggp = _merge_parts(
            _edge_agg_kernel(n, ech)(*_hparts(t01, 4), *_hparts(be[2], 4),
                                     *_hparts(x, 4),
                                     src_t, dst_t, f0_t, f1_t, f2_t), n)
        ac = lp["ac"]
        x = _gine_tc(x, aggp, ac["eps"].reshape(1, 1), ac["W1"], r2(ac["b1"]),
                     r2(ac["g1"]), r2(ac["be1"]), ac["W2"], r2(ac["b2"]),
                     r2(lp["abn_g"]), r2(lp["abn_b"]))
        sums_a, cnt_a = _seg_sum_kernel(mch, nrg)(*_hparts(x, 2),
                                                  row_g, col_s)
        rg = _segmean_proj_tc(rg0, _merge_parts(sums_a, nrg),
                              cnt_a[:, :nrg],
                              lp["r2g_W"], r2(lp["r2g_b"]), True)
        sums_c, _cnt_c = _seg_sum_kernel(rch, nrg)(*_hparts(rg, 2),
                                                   s2_g, d2_s)
        rc = lp["rc"]
        rg = _gine_tc(rg, _merge_parts(sums_c, nrg),
                      rc["eps"].reshape(1, 1), rc["W1"],
                      r2(rc["b1"]), r2(rc["g1"]), r2(rc["be1"]), rc["W2"],
                      r2(rc["b2"]), r2(lp["rbn_g"]), r2(lp["rbn_b"]))
        sums_e, cnt_e = _seg_sum_kernel(mch, n, 4)(*_hparts(rg, 4),
                                                   col_g, row_s)
        x = _segmean_proj_tc(x, _merge_parts(sums_e, n), cnt_e[:, :n],
                             lp["g2r_W"], r2(lp["g2r_b"]), True)

    batch2 = batch.astype(jnp.int32).reshape(n, 1)
    rg_num2 = rg_num_atoms_0.astype(jnp.int32).reshape(1, g)
    rg_hi2 = jnp.cumsum(rg_num_atoms_0.astype(jnp.int32)).reshape(1, g)
    return _readout_tc(x, batch2, rg0, rg_hi2, rg_num2, params["al_W"],
                       r2(params["al_b"]), params["rl_W"], r2(params["rl_b"]),
                       params["lin_W"], r2(params["lin_b"]))
